# baseline (speedup 1.0000x reference)
import functools

import jax
import jax.numpy as jnp
from jax import lax
from jax.experimental import pallas as pl
from jax.experimental.pallas import tpu as pltpu
from jax.experimental.pallas import tpu_sc as plsc

B = 16384
N_MODES = 16
TYPES = 3
D_OBS = 10
D_H = 32
D_E = 8
D_Z1 = TYPES * D_H
D_A = D_Z1 + D_H
D_Z2 = TYPES * 16
D_Z3 = TYPES * N_MODES

PK = 128
VS0 = 16
WSLAB = 257
BSLAB = 9
ZROW = 17
AROW = 3

NC = 2
NS = 16
L = 16
NW = NC * NS
TPW = B // NW
VPW = TPW // L

BT = 4096


def _tc_body(obs_ref, wz1_ref, bz1_ref, wz2_ref, bz2_ref, wz3_ref, bz3_ref,
             wr_ref, br_ref, z_ref, pk_ref,
             wa_s, ba_s, w2_s, b2_s, w3_s):
    @pl.when(pl.program_id(0) == 0)
    def _prep():
        w2_s[...] = jnp.zeros((D_A, D_Z2), jnp.float32)
        for t in range(TYPES):
            wa_s[:, t * D_H:(t + 1) * D_H] = wz1_ref[t]
            ba_s[:, t * D_H:(t + 1) * D_H] = bz1_ref[t][None]
            w2_s[t * D_H:(t + 1) * D_H, 16 * t:16 * (t + 1)] = wz2_ref[t]
            b2_s[:, 16 * t:16 * (t + 1)] = bz2_ref[t][None]
            w3_s[16 * t:16 * (t + 1), :] = wz3_ref[t]
        wa_s[:, D_Z1:D_A] = wr_ref[...]
        ba_s[:, D_Z1:D_A] = br_ref[...]

    obs = obs_ref[...]
    tidi = obs[:, 8:9].astype(jnp.int32)
    a = jnp.maximum(
        jnp.dot(obs, wa_s[...], preferred_element_type=jnp.float32)
        + ba_s[...], 0.0)
    h2 = jnp.maximum(
        jnp.dot(a, w2_s[...], preferred_element_type=jnp.float32)
        + b2_s[...], 0.0)
    t48 = lax.broadcasted_iota(jnp.int32, (BT, D_Z2), 1) // 16
    h2m = jnp.where(t48 == tidi, h2, 0.0)
    t3 = lax.broadcasted_iota(jnp.int32, (BT, TYPES), 1)
    onehot = (t3 == tidi).astype(jnp.float32)
    z = (jnp.dot(h2m, w3_s[...], preferred_element_type=jnp.float32)
         + jnp.dot(onehot, bz3_ref[...],
                   preferred_element_type=jnp.float32))
    z_ref[...] = z
    pk_ref[:, 0:N_MODES] = z
    pk_ref[:, VS0:VS0 + D_H] = a[:, D_Z1:D_A]


def _tc_stage(obs, wz1, bz1, wz2, bz2, wz3, bz3, wr, br):
    rep2 = lambda shape: pl.BlockSpec(shape, lambda i: (0, 0))
    rep3 = lambda shape: pl.BlockSpec(shape, lambda i: (0, 0, 0))
    return pl.pallas_call(
        _tc_body,
        grid=(B // BT,),
        in_specs=[
            pl.BlockSpec((BT, D_OBS), lambda i: (i, 0)),
            rep3((TYPES, D_OBS, D_H)), rep2((TYPES, D_H)),
            rep3((TYPES, D_H, 16)), rep2((TYPES, 16)),
            rep3((TYPES, 16, N_MODES)), rep2((TYPES, N_MODES)),
            rep2((D_OBS, D_H)), rep2((1, D_H)),
        ],
        out_specs=[
            pl.BlockSpec((BT, N_MODES), lambda i: (i, 0)),
            pl.BlockSpec((BT, PK), lambda i: (i, 0)),
        ],
        out_shape=[
            jax.ShapeDtypeStruct((B, N_MODES), jnp.float32),
            jax.ShapeDtypeStruct((B, PK), jnp.float32),
        ],
        scratch_shapes=[
            pltpu.VMEM((D_OBS, D_A), jnp.float32),
            pltpu.VMEM((1, D_A), jnp.float32),
            pltpu.VMEM((D_A, D_Z2), jnp.float32),
            pltpu.VMEM((1, D_Z2), jnp.float32),
            pltpu.VMEM((D_Z2, N_MODES), jnp.float32),
        ],
    )(obs, wz1, bz1, wz2, bz2, wz3, bz3, wr, br)


def _sc_body(pk_hbm, wx1_h, bx1_h, wx2_h, bx2_h, wy1_h, by1_h, wy2_h, by2_h,
             act_hbm,
             pk_v, wx1_v, bx1_v, wx2_v, bx2_v, wy1_v, by1_v, wy2_v, by2_v,
             apad_v, act_v, sem):
    wid = lax.axis_index("s") * NC + lax.axis_index("c")
    base = wid * TPW
    copies = [
        pltpu.async_copy(pk_hbm.at[pl.ds(base * PK, TPW * PK)], pk_v, sem),
        pltpu.async_copy(wx1_h, wx1_v, sem),
        pltpu.async_copy(bx1_h, bx1_v, sem),
        pltpu.async_copy(wx2_h, wx2_v, sem),
        pltpu.async_copy(bx2_h, bx2_v, sem),
        pltpu.async_copy(wy1_h, wy1_v, sem),
        pltpu.async_copy(by1_h, by1_v, sem),
        pltpu.async_copy(wy2_h, wy2_v, sem),
        pltpu.async_copy(by2_h, by2_v, sem),
    ]
    for cp in copies:
        cp.wait()

    lane = lax.iota(jnp.int32, L)
    neg_inf = jnp.full((L,), -jnp.inf, jnp.float32)

    def per_vreg(v, c):
        tok = lane + v * L
        rb = tok * PK
        best_val = neg_inf
        best = jnp.zeros((L,), jnp.int32)
        for m in range(N_MODES):
            zm = plsc.load_gather(pk_v, [rb + m])
            gt = zm > best_val
            best_val = jnp.where(gt, zm, best_val)
            best = jnp.where(gt, m, best)

        mb1 = best * WSLAB
        mbb = best * BSLAB
        vsb = rb + VS0

        def head_pass(w1_v, b1_v):
            h = tuple(plsc.load_gather(b1_v, [mbb + k]) for k in range(D_E))

            def dstep(d4, h):
                for dd in range(4):
                    d = d4 * 4 + dd
                    vsd = plsc.load_gather(pk_v, [vsb + d])
                    wb = mb1 + d * D_E
                    h = tuple(h[k] + vsd * plsc.load_gather(w1_v, [wb + k])
                              for k in range(D_E))
                return h

            return lax.fori_loop(0, D_H // 4, dstep, h)

        hx = head_pass(wx1_v, bx1_v)
        hy = head_pass(wy1_v, by1_v)
        lx = plsc.load_gather(bx2_v, [best])
        ly = plsc.load_gather(by2_v, [best])
        for k in range(D_E):
            lx = lx + jnp.maximum(hx[k], 0.0) * plsc.load_gather(wx2_v, [mbb + k])
            ly = ly + jnp.maximum(hy[k], 0.0) * plsc.load_gather(wy2_v, [mbb + k])
        ra = tok * AROW
        plsc.store_scatter(apad_v, [ra], lx)
        plsc.store_scatter(apad_v, [ra + 1], ly)
        return c

    lax.fori_loop(0, VPW, per_vreg, 0)

    apat = (lane // 2) * AROW + (lane % 2)

    def acompact(g, c):
        pair = plsc.load_gather(apad_v, [g * (8 * AROW) + apat])
        act_v[pl.ds(g * L, L)] = pair
        return c

    lax.fori_loop(0, TPW * 2 // L, acompact, 0)

    pltpu.sync_copy(act_v, act_hbm.at[pl.ds(base * 2, TPW * 2)])


_SC_SCRATCH = [
    pltpu.VMEM((TPW * PK,), jnp.float32),
    pltpu.VMEM((N_MODES * WSLAB,), jnp.float32),
    pltpu.VMEM((N_MODES * BSLAB,), jnp.float32),
    pltpu.VMEM((N_MODES * BSLAB,), jnp.float32),
    pltpu.VMEM((N_MODES,), jnp.float32),
    pltpu.VMEM((N_MODES * WSLAB,), jnp.float32),
    pltpu.VMEM((N_MODES * BSLAB,), jnp.float32),
    pltpu.VMEM((N_MODES * BSLAB,), jnp.float32),
    pltpu.VMEM((N_MODES,), jnp.float32),
    pltpu.VMEM((TPW * AROW,), jnp.float32),
    pltpu.VMEM((TPW * 2,), jnp.float32),
    pltpu.SemaphoreType.DMA,
]


@functools.cache
def _sc_stage_built():
    return functools.partial(
        pl.kernel,
        out_type=jax.ShapeDtypeStruct((B * 2,), jnp.float32),
        mesh=plsc.VectorSubcoreMesh(core_axis_name="c", subcore_axis_name="s",
                                    num_cores=NC, num_subcores=NS),
        scratch_types=_SC_SCRATCH,
        compiler_params=pltpu.CompilerParams(needs_layout_passes=False),
    )(_sc_body)


def _pad_slab(w, slab):
    e, n = w.shape
    return jnp.concatenate(
        [w, jnp.zeros((e, slab - n), w.dtype)], axis=1).reshape(-1)


def kernel(obs_vec, W_root, b_root, Wx1, bx1, Wx2, bx2, Wy1, by1, Wy2, by2,
           Wz1, bz1, Wz2, bz2, Wz3, bz3):
    z_logits, pk = _tc_stage(obs_vec, Wz1, bz1, Wz2, bz2, Wz3, bz3,
                             W_root, b_root.reshape(1, D_H))

    act_flat = _sc_stage_built()(
        pk.reshape(-1),
        _pad_slab(Wx1.reshape(N_MODES, D_H * D_E), WSLAB),
        _pad_slab(bx1, BSLAB),
        _pad_slab(Wx2[:, :, 0], BSLAB), bx2[:, 0],
        _pad_slab(Wy1.reshape(N_MODES, D_H * D_E), WSLAB),
        _pad_slab(by1, BSLAB),
        _pad_slab(Wy2[:, :, 0], BSLAB), by2[:, 0],
    )
    return act_flat.reshape(B, 2), z_logits


# speedup vs baseline: 1.0682x; 1.0682x over previous
import functools

import jax
import jax.numpy as jnp
from jax import lax
from jax.experimental import pallas as pl
from jax.experimental.pallas import tpu as pltpu
from jax.experimental.pallas import tpu_sc as plsc

B = 16384
N_MODES = 16
TYPES = 3
D_OBS = 10
D_H = 32
D_E = 8
D_Z1 = TYPES * D_H
D_A = D_Z1 + D_H
D_Z2 = TYPES * 16
D_Z3 = TYPES * N_MODES

PK = 128
VS0 = 16
WSLAB = 257
BSLAB = 9
ZROW = 17
AROW = 3

NC = 2
NS = 16
L = 16
NW = NC * NS
TPW = B // NW
VPW = TPW // L

BT = 8192


def _tc_body(obs_ref, wz1_ref, bz1_ref, wz2_ref, bz2_ref, wz3_ref, bz3_ref,
             wr_ref, br_ref, z_ref, pk_ref,
             wa_s, ba_s, w2_s, b2_s, w3_s):
    @pl.when(pl.program_id(0) == 0)
    def _prep():
        w2_s[...] = jnp.zeros((D_A, D_Z2), jnp.float32)
        for t in range(TYPES):
            wa_s[:, t * D_H:(t + 1) * D_H] = wz1_ref[t]
            ba_s[:, t * D_H:(t + 1) * D_H] = bz1_ref[t][None]
            w2_s[t * D_H:(t + 1) * D_H, 16 * t:16 * (t + 1)] = wz2_ref[t]
            b2_s[:, 16 * t:16 * (t + 1)] = bz2_ref[t][None]
            w3_s[16 * t:16 * (t + 1), :] = wz3_ref[t]
        wa_s[:, D_Z1:D_A] = wr_ref[...]
        ba_s[:, D_Z1:D_A] = br_ref[...]

    obs = obs_ref[...]
    tidi = obs[:, 8:9].astype(jnp.int32)
    a = jnp.maximum(
        jnp.dot(obs, wa_s[...], preferred_element_type=jnp.float32)
        + ba_s[...], 0.0)
    h2 = jnp.maximum(
        jnp.dot(a, w2_s[...], preferred_element_type=jnp.float32)
        + b2_s[...], 0.0)
    t48 = lax.broadcasted_iota(jnp.int32, (BT, D_Z2), 1) // 16
    h2m = jnp.where(t48 == tidi, h2, 0.0)
    t3 = lax.broadcasted_iota(jnp.int32, (BT, TYPES), 1)
    onehot = (t3 == tidi).astype(jnp.float32)
    z = (jnp.dot(h2m, w3_s[...], preferred_element_type=jnp.float32)
         + jnp.dot(onehot, bz3_ref[...],
                   preferred_element_type=jnp.float32))
    z_ref[...] = z
    pk_ref[:, 0:N_MODES] = z
    pk_ref[:, VS0:VS0 + D_H] = a[:, D_Z1:D_A]


def _tc_stage(obs, wz1, bz1, wz2, bz2, wz3, bz3, wr, br):
    rep2 = lambda shape: pl.BlockSpec(shape, lambda i: (0, 0))
    rep3 = lambda shape: pl.BlockSpec(shape, lambda i: (0, 0, 0))
    return pl.pallas_call(
        _tc_body,
        grid=(B // BT,),
        in_specs=[
            pl.BlockSpec((BT, D_OBS), lambda i: (i, 0)),
            rep3((TYPES, D_OBS, D_H)), rep2((TYPES, D_H)),
            rep3((TYPES, D_H, 16)), rep2((TYPES, 16)),
            rep3((TYPES, 16, N_MODES)), rep2((TYPES, N_MODES)),
            rep2((D_OBS, D_H)), rep2((1, D_H)),
        ],
        out_specs=[
            pl.BlockSpec((BT, N_MODES), lambda i: (i, 0)),
            pl.BlockSpec((BT, PK), lambda i: (i, 0)),
        ],
        out_shape=[
            jax.ShapeDtypeStruct((B, N_MODES), jnp.float32),
            jax.ShapeDtypeStruct((B, PK), jnp.float32),
        ],
        scratch_shapes=[
            pltpu.VMEM((D_OBS, D_A), jnp.float32),
            pltpu.VMEM((1, D_A), jnp.float32),
            pltpu.VMEM((D_A, D_Z2), jnp.float32),
            pltpu.VMEM((1, D_Z2), jnp.float32),
            pltpu.VMEM((D_Z2, N_MODES), jnp.float32),
        ],
    )(obs, wz1, bz1, wz2, bz2, wz3, bz3, wr, br)


def _sc_body(pk_hbm, wx1_h, bx1_h, wx2_h, bx2_h, wy1_h, by1_h, wy2_h, by2_h,
             act_hbm,
             pk_v, wx1_v, bx1_v, wx2_v, bx2_v, wy1_v, by1_v, wy2_v, by2_v,
             apad_v, act_v, sem):
    wid = lax.axis_index("s") * NC + lax.axis_index("c")
    base = wid * TPW
    copies = [
        pltpu.async_copy(pk_hbm.at[pl.ds(base * PK, TPW * PK)], pk_v, sem),
        pltpu.async_copy(wx1_h, wx1_v, sem),
        pltpu.async_copy(bx1_h, bx1_v, sem),
        pltpu.async_copy(wx2_h, wx2_v, sem),
        pltpu.async_copy(bx2_h, bx2_v, sem),
        pltpu.async_copy(wy1_h, wy1_v, sem),
        pltpu.async_copy(by1_h, by1_v, sem),
        pltpu.async_copy(wy2_h, wy2_v, sem),
        pltpu.async_copy(by2_h, by2_v, sem),
    ]
    for cp in copies:
        cp.wait()

    lane = lax.iota(jnp.int32, L)
    neg_inf = jnp.full((L,), -jnp.inf, jnp.float32)

    def per_vreg(v, c):
        tok = lane + v * L
        rb = tok * PK
        best_val = neg_inf
        best = jnp.zeros((L,), jnp.int32)
        for m in range(N_MODES):
            zm = plsc.load_gather(pk_v, [rb + m])
            gt = zm > best_val
            best_val = jnp.where(gt, zm, best_val)
            best = jnp.where(gt, m, best)

        mb1 = best * WSLAB
        mbb = best * BSLAB
        hx = tuple(plsc.load_gather(bx1_v, [mbb + k]) for k in range(D_E))
        hy = tuple(plsc.load_gather(by1_v, [mbb + k]) for k in range(D_E))
        vsb = rb + VS0

        def dstep(d8, carry):
            hx, hy = carry
            for dd in range(4):
                d = d8 * 4 + dd
                vsd = plsc.load_gather(pk_v, [vsb + d])
                wb = mb1 + d * D_E
                hx = tuple(hx[k] + vsd * plsc.load_gather(wx1_v, [wb + k])
                           for k in range(D_E))
                hy = tuple(hy[k] + vsd * plsc.load_gather(wy1_v, [wb + k])
                           for k in range(D_E))
            return hx, hy

        hx, hy = lax.fori_loop(0, D_H // 4, dstep, (hx, hy))
        lx = plsc.load_gather(bx2_v, [best])
        ly = plsc.load_gather(by2_v, [best])
        for k in range(D_E):
            lx = lx + jnp.maximum(hx[k], 0.0) * plsc.load_gather(wx2_v, [mbb + k])
            ly = ly + jnp.maximum(hy[k], 0.0) * plsc.load_gather(wy2_v, [mbb + k])
        ra = tok * AROW
        plsc.store_scatter(apad_v, [ra], lx)
        plsc.store_scatter(apad_v, [ra + 1], ly)
        return c

    lax.fori_loop(0, VPW, per_vreg, 0)

    apat = (lane // 2) * AROW + (lane % 2)

    def acompact(g, c):
        pair = plsc.load_gather(apad_v, [g * (8 * AROW) + apat])
        act_v[pl.ds(g * L, L)] = pair
        return c

    lax.fori_loop(0, TPW * 2 // L, acompact, 0)

    pltpu.sync_copy(act_v, act_hbm.at[pl.ds(base * 2, TPW * 2)])


_SC_SCRATCH = [
    pltpu.VMEM((TPW * PK,), jnp.float32),
    pltpu.VMEM((N_MODES * WSLAB,), jnp.float32),
    pltpu.VMEM((N_MODES * BSLAB,), jnp.float32),
    pltpu.VMEM((N_MODES * BSLAB,), jnp.float32),
    pltpu.VMEM((N_MODES,), jnp.float32),
    pltpu.VMEM((N_MODES * WSLAB,), jnp.float32),
    pltpu.VMEM((N_MODES * BSLAB,), jnp.float32),
    pltpu.VMEM((N_MODES * BSLAB,), jnp.float32),
    pltpu.VMEM((N_MODES,), jnp.float32),
    pltpu.VMEM((TPW * AROW,), jnp.float32),
    pltpu.VMEM((TPW * 2,), jnp.float32),
    pltpu.SemaphoreType.DMA,
]


@functools.cache
def _sc_stage_built():
    return functools.partial(
        pl.kernel,
        out_type=jax.ShapeDtypeStruct((B * 2,), jnp.float32),
        mesh=plsc.VectorSubcoreMesh(core_axis_name="c", subcore_axis_name="s",
                                    num_cores=NC, num_subcores=NS),
        scratch_types=_SC_SCRATCH,
        compiler_params=pltpu.CompilerParams(needs_layout_passes=False),
    )(_sc_body)


def _pad_slab(w, slab):
    e, n = w.shape
    return jnp.concatenate(
        [w, jnp.zeros((e, slab - n), w.dtype)], axis=1).reshape(-1)


def kernel(obs_vec, W_root, b_root, Wx1, bx1, Wx2, bx2, Wy1, by1, Wy2, by2,
           Wz1, bz1, Wz2, bz2, Wz3, bz3):
    z_logits, pk = _tc_stage(obs_vec, Wz1, bz1, Wz2, bz2, Wz3, bz3,
                             W_root, b_root.reshape(1, D_H))

    act_flat = _sc_stage_built()(
        pk.reshape(-1),
        _pad_slab(Wx1.reshape(N_MODES, D_H * D_E), WSLAB),
        _pad_slab(bx1, BSLAB),
        _pad_slab(Wx2[:, :, 0], BSLAB), bx2[:, 0],
        _pad_slab(Wy1.reshape(N_MODES, D_H * D_E), WSLAB),
        _pad_slab(by1, BSLAB),
        _pad_slab(Wy2[:, :, 0], BSLAB), by2[:, 0],
    )
    return act_flat.reshape(B, 2), z_logits
